# baseline (speedup 1.0000x reference)
import dataclasses
import functools

import jax
import jax.numpy as jnp
from jax import lax
from jax.experimental import pallas as pl
from jax.experimental.pallas import tpu as pltpu
from jax.experimental.pallas import tpu_sc as plsc

N_SUITS = 4
N_RANKS = 13
VOCAB = N_SUITS * N_RANKS
DIM = 128
NUM_CARDS = 20
BATCH = 16384
CWIDTH = 128

NUM_CORES = 2
NUM_SUBCORES = 16
NUM_TILES = NUM_CORES * NUM_SUBCORES
ROWS_PER_TILE = BATCH // NUM_TILES
LANES = 16
GROUPS = ROWS_PER_TILE // LANES

X_WORDS = ROWS_PER_TILE * NUM_CARDS
CNT_WORDS = ROWS_PER_TILE * CWIDTH


def _sc_histogram(x):
  mesh = plsc.VectorSubcoreMesh(
      core_axis_name="c",
      subcore_axis_name="s",
      num_cores=NUM_CORES,
      num_subcores=NUM_SUBCORES,
  )

  cp = pltpu.CompilerParams()
  if "needs_layout_passes" in pltpu.CompilerParams.__dataclass_fields__:
    cp = dataclasses.replace(cp, needs_layout_passes=False)
  cp = dataclasses.replace(cp, use_tc_tiling_on_sc=False)

  slabs = 4
  slab_rows = ROWS_PER_TILE // slabs
  slab_groups = slab_rows // LANES

  @functools.partial(
      pl.kernel,
      out_type=jax.ShapeDtypeStruct((BATCH, CWIDTH), jnp.float32),
      mesh=mesh,
      compiler_params=cp,
      scratch_types=[
          pltpu.VMEM((ROWS_PER_TILE, NUM_CARDS), jnp.int32),
          pltpu.VMEM((ROWS_PER_TILE, CWIDTH), jnp.float32),
          pltpu.SemaphoreType.DMA,
          pltpu.SemaphoreType.DMA,
          pltpu.SemaphoreType.DMA,
          pltpu.SemaphoreType.DMA,
          pltpu.SemaphoreType.DMA,
      ],
  )
  def hist_kernel(x_hbm, counts_hbm, x_loc, cnt_loc, s0, s1, s2, s3, osem):
    wid = lax.axis_index("s") * NUM_CORES + lax.axis_index("c")
    base = wid * ROWS_PER_TILE
    xsems = [s0, s1, s2, s3]

    x_dmas = [
        pltpu.async_copy(
            x_hbm.at[pl.ds(base + s * slab_rows, slab_rows), :],
            x_loc.at[pl.ds(s * slab_rows, slab_rows), :],
            xsems[s],
        )
        for s in range(slabs)
    ]

    zeros = jnp.zeros((LANES,), jnp.float32)

    def zero_slab(s):
      @pl.loop(s * slab_rows, (s + 1) * slab_rows)
      def _(r):
        for k in range(CWIDTH // LANES):
          cnt_loc[r, pl.ds(k * LANES, LANES)] = zeros

    lane = lax.iota(jnp.int32, LANES)
    zero_i = jnp.zeros((LANES,), jnp.int32)
    ones = jnp.ones((LANES,), jnp.float32)

    def hist_slab(s):
      @pl.loop(s * slab_groups, (s + 1) * slab_groups, step=4)
      def _(g):
        rows = [g * LANES + k * LANES + lane for k in range(4)]
        for j in range(NUM_CARDS):
          col = zero_i + j
          xvs = [plsc.load_gather(x_loc, [r, col]) for r in rows]
          for r, xv in zip(rows, xvs):
            plsc.addupdate_scatter(cnt_loc, [r, xv], ones)

    zero_slab(0)
    out_dmas = []
    for s in range(slabs):
      x_dmas[s].wait()
      hist_slab(s)
      if s + 1 < slabs:
        zero_slab(s + 1)
      out_dmas.append(
          pltpu.async_copy(
              cnt_loc.at[pl.ds(s * slab_rows, slab_rows), :],
              counts_hbm.at[pl.ds(base + s * slab_rows, slab_rows), :],
              osem,
          )
      )
    for d in out_dmas:
      d.wait()

  return hist_kernel(x)


def _tc_matmul(counts, card_w, rank_w, suit_w):
  blk = 2048
  grid = (BATCH // blk,)

  def body(cnt_ref, card_ref, rank_ref, suit_ref, out_ref, hi_ref, lo_ref):
    @pl.when(pl.program_id(0) == 0)
    def _():
      c_card = lax.broadcasted_iota(jnp.int32, (CWIDTH, VOCAB), 0)
      v_card = lax.broadcasted_iota(jnp.int32, (CWIDTH, VOCAB), 1)
      card_oh = (c_card == v_card).astype(jnp.float32)
      c_rank = lax.broadcasted_iota(jnp.int32, (CWIDTH, N_RANKS), 0)
      v_rank = lax.broadcasted_iota(jnp.int32, (CWIDTH, N_RANKS), 1)
      rank_oh = (c_rank // N_SUITS == v_rank).astype(jnp.float32)
      c_suit = lax.broadcasted_iota(jnp.int32, (CWIDTH, N_SUITS), 0)
      v_suit = lax.broadcasted_iota(jnp.int32, (CWIDTH, N_SUITS), 1)
      suit_oh = ((c_suit % N_SUITS == v_suit) & (c_suit < VOCAB)).astype(
          jnp.float32
      )
      comb = (
          jnp.dot(card_oh, card_ref[...], preferred_element_type=jnp.float32)
          + jnp.dot(rank_oh, rank_ref[...], preferred_element_type=jnp.float32)
          + jnp.dot(suit_oh, suit_ref[...], preferred_element_type=jnp.float32)
      )
      hi = comb.astype(jnp.bfloat16)
      hi_ref[...] = hi
      lo_ref[...] = (comb - hi.astype(jnp.float32)).astype(jnp.bfloat16)

    cnt16 = cnt_ref[...].astype(jnp.bfloat16)
    out_ref[...] = jnp.dot(
        cnt16, hi_ref[...], preferred_element_type=jnp.float32
    ) + jnp.dot(cnt16, lo_ref[...], preferred_element_type=jnp.float32)

  return pl.pallas_call(
      body,
      grid=grid,
      in_specs=[
          pl.BlockSpec((blk, CWIDTH), lambda i: (i, 0)),
          pl.BlockSpec((VOCAB, DIM), lambda i: (0, 0)),
          pl.BlockSpec((N_RANKS, DIM), lambda i: (0, 0)),
          pl.BlockSpec((N_SUITS, DIM), lambda i: (0, 0)),
      ],
      out_specs=pl.BlockSpec((blk, DIM), lambda i: (i, 0)),
      out_shape=jax.ShapeDtypeStruct((BATCH, DIM), jnp.float32),
      scratch_shapes=[
          pltpu.VMEM((CWIDTH, DIM), jnp.bfloat16),
          pltpu.VMEM((CWIDTH, DIM), jnp.bfloat16),
      ],
  )(counts, card_w, rank_w, suit_w)


@jax.jit
def kernel(x, card_w, rank_w, suit_w):
  counts = _sc_histogram(x)
  return _tc_matmul(counts, card_w, rank_w, suit_w)


# speedup vs baseline: 1.0679x; 1.0679x over previous
import dataclasses
import functools

import jax
import jax.numpy as jnp
from jax import lax
from jax.experimental import pallas as pl
from jax.experimental.pallas import tpu as pltpu
from jax.experimental.pallas import tpu_sc as plsc

N_SUITS = 4
N_RANKS = 13
VOCAB = N_SUITS * N_RANKS
DIM = 128
NUM_CARDS = 20
BATCH = 16384
CWIDTH = 128

NUM_CORES = 2
NUM_SUBCORES = 16
NUM_TILES = NUM_CORES * NUM_SUBCORES
ROWS_PER_TILE = BATCH // NUM_TILES
LANES = 16
GROUPS = ROWS_PER_TILE // LANES

X_WORDS = ROWS_PER_TILE * NUM_CARDS
CNT_WORDS = ROWS_PER_TILE * CWIDTH


def _sc_histogram(x):
  mesh = plsc.VectorSubcoreMesh(
      core_axis_name="c",
      subcore_axis_name="s",
      num_cores=NUM_CORES,
      num_subcores=NUM_SUBCORES,
  )

  cp = pltpu.CompilerParams()
  if "needs_layout_passes" in pltpu.CompilerParams.__dataclass_fields__:
    cp = dataclasses.replace(cp, needs_layout_passes=False)

  slabs = 4
  slab_rows = ROWS_PER_TILE // slabs
  slab_groups = slab_rows // LANES

  @functools.partial(
      pl.kernel,
      out_type=jax.ShapeDtypeStruct((BATCH, CWIDTH), jnp.float32),
      mesh=mesh,
      compiler_params=cp,
      scratch_types=[
          pltpu.VMEM((ROWS_PER_TILE, NUM_CARDS), jnp.int32),
          pltpu.VMEM((ROWS_PER_TILE, CWIDTH), jnp.float32),
          pltpu.SemaphoreType.DMA,
          pltpu.SemaphoreType.DMA,
          pltpu.SemaphoreType.DMA,
          pltpu.SemaphoreType.DMA,
          pltpu.SemaphoreType.DMA,
      ],
  )
  def hist_kernel(x_hbm, counts_hbm, x_loc, cnt_loc, s0, s1, s2, s3, osem):
    wid = lax.axis_index("s") * NUM_CORES + lax.axis_index("c")
    base = wid * ROWS_PER_TILE
    xsems = [s0, s1, s2, s3]

    x_dmas = [
        pltpu.async_copy(
            x_hbm.at[pl.ds(base + s * slab_rows, slab_rows), :],
            x_loc.at[pl.ds(s * slab_rows, slab_rows), :],
            xsems[s],
        )
        for s in range(slabs)
    ]

    zeros = jnp.zeros((LANES,), jnp.float32)

    def zero_slab(s):
      @pl.loop(s * slab_rows, (s + 1) * slab_rows)
      def _(r):
        for k in range(CWIDTH // LANES):
          cnt_loc[r, pl.ds(k * LANES, LANES)] = zeros

    lane = lax.iota(jnp.int32, LANES)
    zero_i = jnp.zeros((LANES,), jnp.int32)
    ones = jnp.ones((LANES,), jnp.float32)

    def hist_slab(s):
      @pl.loop(s * slab_groups, (s + 1) * slab_groups, step=4)
      def _(g):
        rows = [g * LANES + k * LANES + lane for k in range(4)]
        for j in range(NUM_CARDS):
          col = zero_i + j
          xvs = [plsc.load_gather(x_loc, [r, col]) for r in rows]
          for r, xv in zip(rows, xvs):
            plsc.addupdate_scatter(cnt_loc, [r, xv], ones)

    zero_slab(0)
    out_dmas = []
    for s in range(slabs):
      x_dmas[s].wait()
      hist_slab(s)
      if s + 1 < slabs:
        zero_slab(s + 1)
      out_dmas.append(
          pltpu.async_copy(
              cnt_loc.at[pl.ds(s * slab_rows, slab_rows), :],
              counts_hbm.at[pl.ds(base + s * slab_rows, slab_rows), :],
              osem,
          )
      )
    for d in out_dmas:
      d.wait()

  return hist_kernel(x)


def _tc_comb(card_w, rank_w, suit_w):
  def body(card_ref, rank_ref, suit_ref, hi_ref, lo_ref):
    c_card = lax.broadcasted_iota(jnp.int32, (CWIDTH, VOCAB), 0)
    v_card = lax.broadcasted_iota(jnp.int32, (CWIDTH, VOCAB), 1)
    card_oh = (c_card == v_card).astype(jnp.float32)
    c_rank = lax.broadcasted_iota(jnp.int32, (CWIDTH, N_RANKS), 0)
    v_rank = lax.broadcasted_iota(jnp.int32, (CWIDTH, N_RANKS), 1)
    rank_oh = (c_rank // N_SUITS == v_rank).astype(jnp.float32)
    c_suit = lax.broadcasted_iota(jnp.int32, (CWIDTH, N_SUITS), 0)
    v_suit = lax.broadcasted_iota(jnp.int32, (CWIDTH, N_SUITS), 1)
    suit_oh = ((c_suit % N_SUITS == v_suit) & (c_suit < VOCAB)).astype(
        jnp.float32
    )
    comb = (
        jnp.dot(card_oh, card_ref[...], preferred_element_type=jnp.float32)
        + jnp.dot(rank_oh, rank_ref[...], preferred_element_type=jnp.float32)
        + jnp.dot(suit_oh, suit_ref[...], preferred_element_type=jnp.float32)
    )
    hi = comb.astype(jnp.bfloat16)
    hi_ref[...] = hi
    lo_ref[...] = (comb - hi.astype(jnp.float32)).astype(jnp.bfloat16)

  return pl.pallas_call(
      body,
      out_shape=(
          jax.ShapeDtypeStruct((CWIDTH, DIM), jnp.bfloat16),
          jax.ShapeDtypeStruct((CWIDTH, DIM), jnp.bfloat16),
      ),
  )(card_w, rank_w, suit_w)


def _tc_matmul(counts, hi, lo):
  blk = 1024
  grid = (BATCH // blk,)

  def body(cnt_ref, hi_ref, lo_ref, out_ref):
    cnt16 = cnt_ref[...].astype(jnp.bfloat16)
    out_ref[...] = jnp.dot(
        cnt16, hi_ref[...], preferred_element_type=jnp.float32
    ) + jnp.dot(cnt16, lo_ref[...], preferred_element_type=jnp.float32)

  return pl.pallas_call(
      body,
      grid=grid,
      in_specs=[
          pl.BlockSpec((blk, CWIDTH), lambda i: (i, 0)),
          pl.BlockSpec((CWIDTH, DIM), lambda i: (0, 0)),
          pl.BlockSpec((CWIDTH, DIM), lambda i: (0, 0)),
      ],
      out_specs=pl.BlockSpec((blk, DIM), lambda i: (i, 0)),
      out_shape=jax.ShapeDtypeStruct((BATCH, DIM), jnp.float32),
  )(counts, hi, lo)


@jax.jit
def kernel(x, card_w, rank_w, suit_w):
  hi, lo = _tc_comb(card_w, rank_w, suit_w)
  counts = _sc_histogram(x)
  return _tc_matmul(counts, hi, lo)


# speedup vs baseline: 1.1489x; 1.0758x over previous
import dataclasses
import functools

import jax
import jax.numpy as jnp
from jax import lax
from jax.experimental import pallas as pl
from jax.experimental.pallas import tpu as pltpu
from jax.experimental.pallas import tpu_sc as plsc

N_SUITS = 4
N_RANKS = 13
VOCAB = N_SUITS * N_RANKS
DIM = 128
NUM_CARDS = 20
BATCH = 16384
CWIDTH = 128

NUM_CORES = 2
NUM_SUBCORES = 16
NUM_TILES = NUM_CORES * NUM_SUBCORES
ROWS_PER_TILE = BATCH // NUM_TILES
LANES = 16
GROUPS = ROWS_PER_TILE // LANES

X_WORDS = ROWS_PER_TILE * NUM_CARDS
CNT_WORDS = ROWS_PER_TILE * CWIDTH


def _sc_histogram(x):
  mesh = plsc.VectorSubcoreMesh(
      core_axis_name="c",
      subcore_axis_name="s",
      num_cores=NUM_CORES,
      num_subcores=NUM_SUBCORES,
  )

  cp = pltpu.CompilerParams()
  if "needs_layout_passes" in pltpu.CompilerParams.__dataclass_fields__:
    cp = dataclasses.replace(cp, needs_layout_passes=False)

  slabs = 4
  slab_rows = ROWS_PER_TILE // slabs
  slab_groups = slab_rows // LANES

  @functools.partial(
      pl.kernel,
      out_type=jax.ShapeDtypeStruct((BATCH, CWIDTH), jnp.float32),
      mesh=mesh,
      compiler_params=cp,
      scratch_types=[
          pltpu.VMEM((ROWS_PER_TILE, NUM_CARDS), jnp.int32),
          pltpu.VMEM((ROWS_PER_TILE, CWIDTH), jnp.float32),
          pltpu.SemaphoreType.DMA,
          pltpu.SemaphoreType.DMA,
          pltpu.SemaphoreType.DMA,
          pltpu.SemaphoreType.DMA,
          pltpu.SemaphoreType.DMA,
      ],
  )
  def hist_kernel(x_hbm, counts_hbm, x_loc, cnt_loc, s0, s1, s2, s3, osem):
    wid = lax.axis_index("s") * NUM_CORES + lax.axis_index("c")
    base = wid * ROWS_PER_TILE
    xsems = [s0, s1, s2, s3]

    x_dmas = [
        pltpu.async_copy(
            x_hbm.at[pl.ds(base + s * slab_rows, slab_rows), :],
            x_loc.at[pl.ds(s * slab_rows, slab_rows), :],
            xsems[s],
        )
        for s in range(slabs)
    ]

    zeros = jnp.zeros((LANES,), jnp.float32)

    def zero_slab(s):
      @pl.loop(s * slab_rows, (s + 1) * slab_rows)
      def _(r):
        for k in range(CWIDTH // LANES):
          cnt_loc[r, pl.ds(k * LANES, LANES)] = zeros

    lane = lax.iota(jnp.int32, LANES)
    zero_i = jnp.zeros((LANES,), jnp.int32)
    ones = jnp.ones((LANES,), jnp.float32)

    def hist_slab(s):
      @pl.loop(s * slab_groups, (s + 1) * slab_groups, step=4)
      def _(g):
        rows = [g * LANES + k * LANES + lane for k in range(4)]
        for j in range(NUM_CARDS):
          col = zero_i + j
          xvs = [plsc.load_gather(x_loc, [r, col]) for r in rows]
          for r, xv in zip(rows, xvs):
            plsc.addupdate_scatter(cnt_loc, [r, xv], ones)

    zero_slab(0)
    out_dmas = []
    for s in range(slabs):
      x_dmas[s].wait()
      hist_slab(s)
      if s + 1 < slabs:
        zero_slab(s + 1)
      out_dmas.append(
          pltpu.async_copy(
              cnt_loc.at[pl.ds(s * slab_rows, slab_rows), :],
              counts_hbm.at[pl.ds(base + s * slab_rows, slab_rows), :],
              osem,
          )
      )
    for d in out_dmas:
      d.wait()

  return hist_kernel(x)


def _tc_matmul(counts, card_w, rank_w, suit_w):
  blk = 2048
  grid = (BATCH // blk,)

  def body(cnt_ref, card_ref, rank_ref, suit_ref, out_ref, hi_ref, lo_ref):
    @pl.when(pl.program_id(0) == 0)
    def _():
      c_card = lax.broadcasted_iota(jnp.int32, (CWIDTH, VOCAB), 0)
      v_card = lax.broadcasted_iota(jnp.int32, (CWIDTH, VOCAB), 1)
      card_oh = (c_card == v_card).astype(jnp.float32)
      c_rank = lax.broadcasted_iota(jnp.int32, (CWIDTH, N_RANKS), 0)
      v_rank = lax.broadcasted_iota(jnp.int32, (CWIDTH, N_RANKS), 1)
      rank_oh = (c_rank // N_SUITS == v_rank).astype(jnp.float32)
      c_suit = lax.broadcasted_iota(jnp.int32, (CWIDTH, N_SUITS), 0)
      v_suit = lax.broadcasted_iota(jnp.int32, (CWIDTH, N_SUITS), 1)
      suit_oh = ((c_suit % N_SUITS == v_suit) & (c_suit < VOCAB)).astype(
          jnp.float32
      )
      comb = (
          jnp.dot(card_oh, card_ref[...], preferred_element_type=jnp.float32)
          + jnp.dot(rank_oh, rank_ref[...], preferred_element_type=jnp.float32)
          + jnp.dot(suit_oh, suit_ref[...], preferred_element_type=jnp.float32)
      )
      hi = comb.astype(jnp.bfloat16)
      hi_ref[...] = hi
      lo_ref[...] = (comb - hi.astype(jnp.float32)).astype(jnp.bfloat16)

    cnt16 = cnt_ref[...].astype(jnp.bfloat16)
    out_ref[...] = jnp.dot(
        cnt16, hi_ref[...], preferred_element_type=jnp.float32
    ) + jnp.dot(cnt16, lo_ref[...], preferred_element_type=jnp.float32)

  return pl.pallas_call(
      body,
      grid=grid,
      in_specs=[
          pl.BlockSpec((blk, CWIDTH), lambda i: (i, 0)),
          pl.BlockSpec((VOCAB, DIM), lambda i: (0, 0)),
          pl.BlockSpec((N_RANKS, DIM), lambda i: (0, 0)),
          pl.BlockSpec((N_SUITS, DIM), lambda i: (0, 0)),
      ],
      out_specs=pl.BlockSpec((blk, DIM), lambda i: (i, 0)),
      out_shape=jax.ShapeDtypeStruct((BATCH, DIM), jnp.float32),
      scratch_shapes=[
          pltpu.VMEM((CWIDTH, DIM), jnp.bfloat16),
          pltpu.VMEM((CWIDTH, DIM), jnp.bfloat16),
      ],
  )(counts, card_w, rank_w, suit_w)


@jax.jit
def kernel(x, card_w, rank_w, suit_w):
  counts = _sc_histogram(x)
  return _tc_matmul(counts, card_w, rank_w, suit_w)


# speedup vs baseline: 1.2015x; 1.0458x over previous
import dataclasses
import functools

import jax
import jax.numpy as jnp
from jax import lax
from jax.experimental import pallas as pl
from jax.experimental.pallas import tpu as pltpu
from jax.experimental.pallas import tpu_sc as plsc

N_SUITS = 4
N_RANKS = 13
VOCAB = N_SUITS * N_RANKS
DIM = 128
NUM_CARDS = 20
BATCH = 16384
CWIDTH = 128

NUM_CORES = 2
NUM_SUBCORES = 16
NUM_TILES = NUM_CORES * NUM_SUBCORES
ROWS_PER_TILE = BATCH // NUM_TILES
LANES = 16
GROUPS = ROWS_PER_TILE // LANES

X_WORDS = ROWS_PER_TILE * NUM_CARDS
CNT_WORDS = ROWS_PER_TILE * CWIDTH


def _sc_histogram(x):
  mesh = plsc.VectorSubcoreMesh(
      core_axis_name="c",
      subcore_axis_name="s",
      num_cores=NUM_CORES,
      num_subcores=NUM_SUBCORES,
  )

  cp = pltpu.CompilerParams()
  if "needs_layout_passes" in pltpu.CompilerParams.__dataclass_fields__:
    cp = dataclasses.replace(cp, needs_layout_passes=False)

  slabs = 4
  slab_rows = ROWS_PER_TILE // slabs
  slab_groups = slab_rows // LANES

  @functools.partial(
      pl.kernel,
      out_type=jax.ShapeDtypeStruct((BATCH, CWIDTH), jnp.float32),
      mesh=mesh,
      compiler_params=cp,
      scratch_types=[
          pltpu.VMEM((ROWS_PER_TILE, NUM_CARDS), jnp.int32),
          pltpu.VMEM((ROWS_PER_TILE, CWIDTH), jnp.float32),
          pltpu.SemaphoreType.DMA,
          pltpu.SemaphoreType.DMA,
          pltpu.SemaphoreType.DMA,
          pltpu.SemaphoreType.DMA,
          pltpu.SemaphoreType.DMA,
      ],
  )
  def hist_kernel(x_hbm, counts_hbm, x_loc, cnt_loc, s0, s1, s2, s3, osem):
    wid = lax.axis_index("s") * NUM_CORES + lax.axis_index("c")
    base = wid * ROWS_PER_TILE
    xsems = [s0, s1, s2, s3]

    x_dmas = [
        pltpu.async_copy(
            x_hbm.at[pl.ds(base + s * slab_rows, slab_rows), :],
            x_loc.at[pl.ds(s * slab_rows, slab_rows), :],
            xsems[s],
        )
        for s in range(slabs)
    ]

    zeros = jnp.zeros((LANES,), jnp.float32)

    def zero_slab(s):
      @pl.loop(s * slab_rows, (s + 1) * slab_rows)
      def _(r):
        for k in range(CWIDTH // LANES):
          cnt_loc[r, pl.ds(k * LANES, LANES)] = zeros

    lane = lax.iota(jnp.int32, LANES)
    zero_i = jnp.zeros((LANES,), jnp.int32)
    ones = jnp.ones((LANES,), jnp.float32)

    def hist_slab(s):
      @pl.loop(s * slab_groups, (s + 1) * slab_groups, step=4)
      def _(g):
        rows = [g * LANES + k * LANES + lane for k in range(4)]
        for j in range(NUM_CARDS):
          col = zero_i + j
          xvs = [plsc.load_gather(x_loc, [r, col]) for r in rows]
          for r, xv in zip(rows, xvs):
            plsc.addupdate_scatter(cnt_loc, [r, xv], ones)

    zero_slab(0)
    out_dmas = []
    for s in range(slabs):
      x_dmas[s].wait()
      hist_slab(s)
      if s + 1 < slabs:
        zero_slab(s + 1)
      out_dmas.append(
          pltpu.async_copy(
              cnt_loc.at[pl.ds(s * slab_rows, slab_rows), :],
              counts_hbm.at[pl.ds(base + s * slab_rows, slab_rows), :],
              osem,
          )
      )
    for d in out_dmas:
      d.wait()

  return hist_kernel(x)


def _tc_matmul(counts, card_w, rank_w, suit_w):
  blk = 4096
  grid = (BATCH // blk,)

  def body(cnt_ref, card_ref, rank_ref, suit_ref, out_ref, hi_ref, lo_ref):
    @pl.when(pl.program_id(0) == 0)
    def _():
      c_card = lax.broadcasted_iota(jnp.int32, (CWIDTH, VOCAB), 0)
      v_card = lax.broadcasted_iota(jnp.int32, (CWIDTH, VOCAB), 1)
      card_oh = (c_card == v_card).astype(jnp.float32)
      c_rank = lax.broadcasted_iota(jnp.int32, (CWIDTH, N_RANKS), 0)
      v_rank = lax.broadcasted_iota(jnp.int32, (CWIDTH, N_RANKS), 1)
      rank_oh = (c_rank // N_SUITS == v_rank).astype(jnp.float32)
      c_suit = lax.broadcasted_iota(jnp.int32, (CWIDTH, N_SUITS), 0)
      v_suit = lax.broadcasted_iota(jnp.int32, (CWIDTH, N_SUITS), 1)
      suit_oh = ((c_suit % N_SUITS == v_suit) & (c_suit < VOCAB)).astype(
          jnp.float32
      )
      comb = (
          jnp.dot(card_oh, card_ref[...], preferred_element_type=jnp.float32)
          + jnp.dot(rank_oh, rank_ref[...], preferred_element_type=jnp.float32)
          + jnp.dot(suit_oh, suit_ref[...], preferred_element_type=jnp.float32)
      )
      hi = comb.astype(jnp.bfloat16)
      hi_ref[...] = hi
      lo_ref[...] = (comb - hi.astype(jnp.float32)).astype(jnp.bfloat16)

    cnt16 = cnt_ref[...].astype(jnp.bfloat16)
    out_ref[...] = jnp.dot(
        cnt16, hi_ref[...], preferred_element_type=jnp.float32
    ) + jnp.dot(cnt16, lo_ref[...], preferred_element_type=jnp.float32)

  return pl.pallas_call(
      body,
      grid=grid,
      in_specs=[
          pl.BlockSpec((blk, CWIDTH), lambda i: (i, 0)),
          pl.BlockSpec((VOCAB, DIM), lambda i: (0, 0)),
          pl.BlockSpec((N_RANKS, DIM), lambda i: (0, 0)),
          pl.BlockSpec((N_SUITS, DIM), lambda i: (0, 0)),
      ],
      out_specs=pl.BlockSpec((blk, DIM), lambda i: (i, 0)),
      out_shape=jax.ShapeDtypeStruct((BATCH, DIM), jnp.float32),
      scratch_shapes=[
          pltpu.VMEM((CWIDTH, DIM), jnp.bfloat16),
          pltpu.VMEM((CWIDTH, DIM), jnp.bfloat16),
      ],
  )(counts, card_w, rank_w, suit_w)


@jax.jit
def kernel(x, card_w, rank_w, suit_w):
  counts = _sc_histogram(x)
  return _tc_matmul(counts, card_w, rank_w, suit_w)


# speedup vs baseline: 1.2449x; 1.0361x over previous
import dataclasses
import functools

import jax
import jax.numpy as jnp
from jax import lax
from jax.experimental import pallas as pl
from jax.experimental.pallas import tpu as pltpu
from jax.experimental.pallas import tpu_sc as plsc

N_SUITS = 4
N_RANKS = 13
VOCAB = N_SUITS * N_RANKS
DIM = 128
NUM_CARDS = 20
BATCH = 16384
CWIDTH = 128

NUM_CORES = 2
NUM_SUBCORES = 16
NUM_TILES = NUM_CORES * NUM_SUBCORES
ROWS_PER_TILE = BATCH // NUM_TILES
LANES = 16
GROUPS = ROWS_PER_TILE // LANES

X_WORDS = ROWS_PER_TILE * NUM_CARDS
CNT_WORDS = ROWS_PER_TILE * CWIDTH


def _sc_histogram(x):
  mesh = plsc.VectorSubcoreMesh(
      core_axis_name="c",
      subcore_axis_name="s",
      num_cores=NUM_CORES,
      num_subcores=NUM_SUBCORES,
  )

  cp = pltpu.CompilerParams()
  if "needs_layout_passes" in pltpu.CompilerParams.__dataclass_fields__:
    cp = dataclasses.replace(cp, needs_layout_passes=False)

  slabs = 4
  slab_rows = ROWS_PER_TILE // slabs
  slab_groups = slab_rows // LANES

  @functools.partial(
      pl.kernel,
      out_type=jax.ShapeDtypeStruct((BATCH, CWIDTH), jnp.float32),
      mesh=mesh,
      compiler_params=cp,
      scratch_types=[
          pltpu.VMEM((ROWS_PER_TILE, NUM_CARDS), jnp.int32),
          pltpu.VMEM((ROWS_PER_TILE, CWIDTH), jnp.float32),
          pltpu.SemaphoreType.DMA,
          pltpu.SemaphoreType.DMA,
          pltpu.SemaphoreType.DMA,
          pltpu.SemaphoreType.DMA,
          pltpu.SemaphoreType.DMA,
      ],
  )
  def hist_kernel(x_hbm, counts_hbm, x_loc, cnt_loc, s0, s1, s2, s3, osem):
    wid = lax.axis_index("s") * NUM_CORES + lax.axis_index("c")
    base = wid * ROWS_PER_TILE
    xsems = [s0, s1, s2, s3]

    x_dmas = [
        pltpu.async_copy(
            x_hbm.at[pl.ds(base + s * slab_rows, slab_rows), :],
            x_loc.at[pl.ds(s * slab_rows, slab_rows), :],
            xsems[s],
        )
        for s in range(slabs)
    ]

    zeros = jnp.zeros((LANES,), jnp.float32)

    def zero_slab(s):
      @pl.loop(s * slab_rows, (s + 1) * slab_rows)
      def _(r):
        for k in range(CWIDTH // LANES):
          cnt_loc[r, pl.ds(k * LANES, LANES)] = zeros

    lane = lax.iota(jnp.int32, LANES)
    zero_i = jnp.zeros((LANES,), jnp.int32)
    ones = jnp.ones((LANES,), jnp.float32)

    def hist_slab(s):
      @pl.loop(s * slab_groups, (s + 1) * slab_groups, step=4)
      def _(g):
        rows = [g * LANES + k * LANES + lane for k in range(4)]
        for j in range(NUM_CARDS):
          col = zero_i + j
          xvs = [plsc.load_gather(x_loc, [r, col]) for r in rows]
          for r, xv in zip(rows, xvs):
            plsc.addupdate_scatter(cnt_loc, [r, xv], ones)

    zero_slab(0)
    out_dmas = []
    for s in range(slabs):
      x_dmas[s].wait()
      hist_slab(s)
      if s + 1 < slabs:
        zero_slab(s + 1)
      out_dmas.append(
          pltpu.async_copy(
              cnt_loc.at[pl.ds(s * slab_rows, slab_rows), :],
              counts_hbm.at[pl.ds(base + s * slab_rows, slab_rows), :],
              osem,
          )
      )
    for d in out_dmas:
      d.wait()

  return hist_kernel(x)


def _tc_matmul(counts, card_w, rank_w, suit_w):
  blk = 8192
  grid = (BATCH // blk,)

  def body(cnt_ref, card_ref, rank_ref, suit_ref, out_ref, hi_ref, lo_ref):
    @pl.when(pl.program_id(0) == 0)
    def _():
      c_card = lax.broadcasted_iota(jnp.int32, (CWIDTH, VOCAB), 0)
      v_card = lax.broadcasted_iota(jnp.int32, (CWIDTH, VOCAB), 1)
      card_oh = (c_card == v_card).astype(jnp.float32)
      c_rank = lax.broadcasted_iota(jnp.int32, (CWIDTH, N_RANKS), 0)
      v_rank = lax.broadcasted_iota(jnp.int32, (CWIDTH, N_RANKS), 1)
      rank_oh = (c_rank // N_SUITS == v_rank).astype(jnp.float32)
      c_suit = lax.broadcasted_iota(jnp.int32, (CWIDTH, N_SUITS), 0)
      v_suit = lax.broadcasted_iota(jnp.int32, (CWIDTH, N_SUITS), 1)
      suit_oh = ((c_suit % N_SUITS == v_suit) & (c_suit < VOCAB)).astype(
          jnp.float32
      )
      comb = (
          jnp.dot(card_oh, card_ref[...], preferred_element_type=jnp.float32)
          + jnp.dot(rank_oh, rank_ref[...], preferred_element_type=jnp.float32)
          + jnp.dot(suit_oh, suit_ref[...], preferred_element_type=jnp.float32)
      )
      hi = comb.astype(jnp.bfloat16)
      hi_ref[...] = hi
      lo_ref[...] = (comb - hi.astype(jnp.float32)).astype(jnp.bfloat16)

    cnt16 = cnt_ref[...].astype(jnp.bfloat16)
    out_ref[...] = jnp.dot(
        cnt16, hi_ref[...], preferred_element_type=jnp.float32
    ) + jnp.dot(cnt16, lo_ref[...], preferred_element_type=jnp.float32)

  return pl.pallas_call(
      body,
      grid=grid,
      in_specs=[
          pl.BlockSpec((blk, CWIDTH), lambda i: (i, 0)),
          pl.BlockSpec((VOCAB, DIM), lambda i: (0, 0)),
          pl.BlockSpec((N_RANKS, DIM), lambda i: (0, 0)),
          pl.BlockSpec((N_SUITS, DIM), lambda i: (0, 0)),
      ],
      out_specs=pl.BlockSpec((blk, DIM), lambda i: (i, 0)),
      out_shape=jax.ShapeDtypeStruct((BATCH, DIM), jnp.float32),
      scratch_shapes=[
          pltpu.VMEM((CWIDTH, DIM), jnp.bfloat16),
          pltpu.VMEM((CWIDTH, DIM), jnp.bfloat16),
      ],
  )(counts, card_w, rank_w, suit_w)


@jax.jit
def kernel(x, card_w, rank_w, suit_w):
  counts = _sc_histogram(x)
  return _tc_matmul(counts, card_w, rank_w, suit_w)
